# baseline (device time: 33767 ns/iter reference)
import jax
import jax.numpy as jnp
from jax import lax
from jax.experimental import pallas as pl
from jax.experimental.pallas import tpu as pltpu

T = 1024
D = 1024
V_SHARD = 8192
NQ = 4
TQ = T // NQ


def _allreduce_gather(part):

    def body(part_ref, out_ref, a, b, c, d, recv_y, send_sems, recv_sems):
        xi = lax.axis_index("x")
        yi = lax.axis_index("y")
        zi = lax.axis_index("z")
        q = 2 * xi + zi

        x_partner = (1 - xi, yi, zi)
        y_partner = (xi, 1 - yi, zi)
        z_partner = (xi, yi, 1 - zi)

        barrier_sem = pltpu.get_barrier_semaphore()
        for nbr in (x_partner, y_partner, z_partner):
            pl.semaphore_signal(
                barrier_sem, inc=1,
                device_id=nbr, device_id_type=pl.DeviceIdType.MESH,
            )
        pl.semaphore_wait(barrier_sem, 3)

        rdma_y = pltpu.make_async_remote_copy(
            src_ref=part_ref, dst_ref=recv_y,
            send_sem=send_sems.at[0], recv_sem=recv_sems.at[0],
            device_id=y_partner, device_id_type=pl.DeviceIdType.MESH,
        )
        rdma_y.start()
        rdma_y.wait()
        a[...] = part_ref[...] + recv_y[...]

        rdma_x = pltpu.make_async_remote_copy(
            src_ref=a, dst_ref=b,
            send_sem=send_sems.at[1], recv_sem=recv_sems.at[1],
            device_id=x_partner, device_id_type=pl.DeviceIdType.MESH,
        )
        rdma_z = pltpu.make_async_remote_copy(
            src_ref=a, dst_ref=c,
            send_sem=send_sems.at[2], recv_sem=recv_sems.at[2],
            device_id=z_partner, device_id_type=pl.DeviceIdType.MESH,
        )
        rdma_x.start()
        rdma_z.start()
        rdma_x.wait()
        rdma_z.wait()

        h = D // 2
        rdma_bz = pltpu.make_async_remote_copy(
            src_ref=b.at[:, pl.ds(0, h)], dst_ref=d.at[:, pl.ds(0, h)],
            send_sem=send_sems.at[3], recv_sem=recv_sems.at[3],
            device_id=z_partner, device_id_type=pl.DeviceIdType.MESH,
        )
        rdma_cx = pltpu.make_async_remote_copy(
            src_ref=c.at[:, pl.ds(h, h)], dst_ref=d.at[:, pl.ds(h, h)],
            send_sem=send_sems.at[4], recv_sem=recv_sems.at[4],
            device_id=x_partner, device_id_type=pl.DeviceIdType.MESH,
        )
        rdma_bz.start()
        rdma_cx.start()
        rdma_bz.wait()
        rdma_cx.wait()

        q_a = q
        q_b = 2 * (1 - xi) + zi
        q_c = 2 * xi + (1 - zi)
        q_d = 2 * (1 - xi) + (1 - zi)
        out_ref[pl.ds(q_a * TQ, TQ), :] = a[...].astype(jnp.float32)
        out_ref[pl.ds(q_b * TQ, TQ), :] = b[...].astype(jnp.float32)
        out_ref[pl.ds(q_c * TQ, TQ), :] = c[...].astype(jnp.float32)
        out_ref[pl.ds(q_d * TQ, TQ), :] = d[...].astype(jnp.float32)

    return pl.pallas_call(
        body,
        out_shape=jax.ShapeDtypeStruct((T, D), jnp.float32),
        in_specs=[pl.BlockSpec(memory_space=pltpu.VMEM)],
        out_specs=pl.BlockSpec(memory_space=pltpu.VMEM),
        scratch_shapes=[
            pltpu.VMEM((TQ, D), jnp.bfloat16),
            pltpu.VMEM((TQ, D), jnp.bfloat16),
            pltpu.VMEM((TQ, D), jnp.bfloat16),
            pltpu.VMEM((TQ, D), jnp.bfloat16),
            pltpu.VMEM((TQ, D), jnp.bfloat16),
            pltpu.SemaphoreType.DMA((5,)),
            pltpu.SemaphoreType.DMA((5,)),
        ],
        compiler_params=pltpu.CompilerParams(collective_id=0),
    )(part)


def kernel(ids, E):
    xi = lax.axis_index("x")
    yi = lax.axis_index("y")
    zi = lax.axis_index("z")
    q = 2 * xi + zi

    my_ids = lax.dynamic_slice(ids, (q * TQ,), (TQ,))
    local = my_ids - yi * V_SHARD
    in_range = (local >= 0) & (local < V_SHARD)
    rows = jnp.take(E, jnp.clip(local, 0, V_SHARD - 1), axis=0)
    part = jnp.where(in_range[:, None], rows, 0.0).astype(jnp.bfloat16)

    return _allreduce_gather(part)


# device time: 27336 ns/iter; 1.2353x vs baseline; 1.2353x over previous
import jax
import jax.numpy as jnp
from jax import lax
from jax.experimental import pallas as pl
from jax.experimental.pallas import tpu as pltpu

T = 1024
D = 1024
V_SHARD = 8192
NQ = 4
TQ = T // NQ
NC = 4
R = TQ // NC


def _allreduce_gather(part):

    def body(part_ref, out_ref, recv_y, send_sems, recv_sems):
        xi = lax.axis_index("x")
        yi = lax.axis_index("y")
        zi = lax.axis_index("z")

        x_partner = (1 - xi, yi, zi)
        y_partner = (xi, 1 - yi, zi)
        z_partner = (xi, yi, 1 - zi)

        q_a = 2 * xi + zi
        q_b = 2 * (1 - xi) + zi
        q_c = 2 * xi + (1 - zi)
        a0 = q_a * TQ
        b0 = q_b * TQ
        c0 = q_c * TQ

        barrier_sem = pltpu.get_barrier_semaphore()
        for nbr in (x_partner, y_partner, z_partner):
            pl.semaphore_signal(
                barrier_sem, inc=1,
                device_id=nbr, device_id_type=pl.DeviceIdType.MESH,
            )
        pl.semaphore_wait(barrier_sem, 3)

        rdma_y = []
        for c in range(NC):
            r = pltpu.make_async_remote_copy(
                src_ref=part_ref.at[pl.ds(c * R, R), :],
                dst_ref=recv_y.at[pl.ds(c * R, R), :],
                send_sem=send_sems.at[0, c], recv_sem=recv_sems.at[0, c],
                device_id=y_partner, device_id_type=pl.DeviceIdType.MESH,
            )
            r.start()
            rdma_y.append(r)

        rdma_x = []
        rdma_z = []
        for c in range(NC):
            rdma_y[c].wait()
            out_ref[pl.ds(a0 + c * R, R), :] = (
                part_ref[pl.ds(c * R, R), :] + recv_y[pl.ds(c * R, R), :]
            )
            rx = pltpu.make_async_remote_copy(
                src_ref=out_ref.at[pl.ds(a0 + c * R, R), :],
                dst_ref=out_ref.at[pl.ds(a0 + c * R, R), :],
                send_sem=send_sems.at[1, c], recv_sem=recv_sems.at[1, c],
                device_id=x_partner, device_id_type=pl.DeviceIdType.MESH,
            )
            rx.start()
            rdma_x.append(rx)
            rz = pltpu.make_async_remote_copy(
                src_ref=out_ref.at[pl.ds(a0 + c * R, R), :],
                dst_ref=out_ref.at[pl.ds(a0 + c * R, R), :],
                send_sem=send_sems.at[2, c], recv_sem=recv_sems.at[2, c],
                device_id=z_partner, device_id_type=pl.DeviceIdType.MESH,
            )
            rz.start()
            rdma_z.append(rz)

        rdma_f = []
        for c in range(NC):
            rdma_x[c].wait()
            rdma_z[c].wait()
            if c < NC // 2:
                src0 = b0 + c * R
                tgt = z_partner
            else:
                src0 = c0 + c * R
                tgt = x_partner
            rf = pltpu.make_async_remote_copy(
                src_ref=out_ref.at[pl.ds(src0, R), :],
                dst_ref=out_ref.at[pl.ds(src0, R), :],
                send_sem=send_sems.at[3, c], recv_sem=recv_sems.at[3, c],
                device_id=tgt, device_id_type=pl.DeviceIdType.MESH,
            )
            rf.start()
            rdma_f.append(rf)

        for c in range(NC):
            rdma_f[c].wait()

    return pl.pallas_call(
        body,
        out_shape=jax.ShapeDtypeStruct((T, D), jnp.bfloat16),
        in_specs=[pl.BlockSpec(memory_space=pltpu.VMEM)],
        out_specs=pl.BlockSpec(memory_space=pltpu.VMEM),
        scratch_shapes=[
            pltpu.VMEM((TQ, D), jnp.bfloat16),
            pltpu.SemaphoreType.DMA((4, NC)),
            pltpu.SemaphoreType.DMA((4, NC)),
        ],
        compiler_params=pltpu.CompilerParams(collective_id=0),
    )(part)


def kernel(ids, E):
    xi = lax.axis_index("x")
    yi = lax.axis_index("y")
    zi = lax.axis_index("z")
    q = 2 * xi + zi

    my_ids = lax.dynamic_slice(ids, (q * TQ,), (TQ,))
    local = my_ids - yi * V_SHARD
    in_range = (local >= 0) & (local < V_SHARD)
    rows = jnp.take(E, jnp.clip(local, 0, V_SHARD - 1), axis=0)
    part = jnp.where(in_range[:, None], rows, 0.0).astype(jnp.bfloat16)

    return _allreduce_gather(part)


# device time: 25731 ns/iter; 1.3123x vs baseline; 1.0624x over previous
import jax
import jax.numpy as jnp
from jax import lax
from jax.experimental import pallas as pl
from jax.experimental.pallas import tpu as pltpu

T = 1024
D = 1024
V_SHARD = 8192
NQ = 4
TQ = T // NQ
NC = 4
R = TQ // NC


def _fused(idx, maskf, E):

    def body(idx_ref, mask_ref, e_ref, out_ref,
             rows, part, recv_y, gsems, send_sems, recv_sems):
        xi = lax.axis_index("x")
        yi = lax.axis_index("y")
        zi = lax.axis_index("z")

        x_partner = (1 - xi, yi, zi)
        y_partner = (xi, 1 - yi, zi)
        z_partner = (xi, yi, 1 - zi)

        q_a = 2 * xi + zi
        q_b = 2 * (1 - xi) + zi
        q_c = 2 * xi + (1 - zi)
        a0 = q_a * TQ
        b0 = q_b * TQ
        c0 = q_c * TQ

        def issue_row(i, _):
            pltpu.make_async_copy(
                e_ref.at[pl.ds(idx_ref[i], 1), :],
                rows.at[pl.ds(i, 1), :],
                gsems.at[i // R],
            ).start()
            return 0

        lax.fori_loop(0, TQ, issue_row, 0, unroll=8)

        barrier_sem = pltpu.get_barrier_semaphore()
        for nbr in (x_partner, y_partner, z_partner):
            pl.semaphore_signal(
                barrier_sem, inc=1,
                device_id=nbr, device_id_type=pl.DeviceIdType.MESH,
            )
        pl.semaphore_wait(barrier_sem, 3)

        rdma_y = []
        for c in range(NC):
            def wait_row(i, _, c=c):
                pltpu.make_async_copy(
                    e_ref.at[pl.ds(0, 1), :],
                    rows.at[pl.ds(i, 1), :],
                    gsems.at[c],
                ).wait()
                return 0

            lax.fori_loop(c * R, (c + 1) * R, wait_row, 0, unroll=8)
            sl = pl.ds(c * R, R)
            part[sl, :] = (rows[sl, :] * mask_ref[sl, :]).astype(jnp.bfloat16)
            r = pltpu.make_async_remote_copy(
                src_ref=part.at[sl, :],
                dst_ref=recv_y.at[sl, :],
                send_sem=send_sems.at[0, c], recv_sem=recv_sems.at[0, c],
                device_id=y_partner, device_id_type=pl.DeviceIdType.MESH,
            )
            r.start()
            rdma_y.append(r)

        rdma_x = []
        rdma_z = []
        for c in range(NC):
            rdma_y[c].wait()
            sl = pl.ds(c * R, R)
            out_ref[pl.ds(a0 + c * R, R), :] = part[sl, :] + recv_y[sl, :]
            for k, tgt, lst in ((1, x_partner, rdma_x), (2, z_partner, rdma_z)):
                r = pltpu.make_async_remote_copy(
                    src_ref=out_ref.at[pl.ds(a0 + c * R, R), :],
                    dst_ref=out_ref.at[pl.ds(a0 + c * R, R), :],
                    send_sem=send_sems.at[k, c], recv_sem=recv_sems.at[k, c],
                    device_id=tgt, device_id_type=pl.DeviceIdType.MESH,
                )
                r.start()
                lst.append(r)

        rdma_f = []
        for c in range(NC):
            rdma_x[c].wait()
            rdma_z[c].wait()
            if c < NC // 2:
                src0, tgt = b0 + c * R, z_partner
            else:
                src0, tgt = c0 + c * R, x_partner
            r = pltpu.make_async_remote_copy(
                src_ref=out_ref.at[pl.ds(src0, R), :],
                dst_ref=out_ref.at[pl.ds(src0, R), :],
                send_sem=send_sems.at[3, c], recv_sem=recv_sems.at[3, c],
                device_id=tgt, device_id_type=pl.DeviceIdType.MESH,
            )
            r.start()
            rdma_f.append(r)

        for c in range(NC):
            rdma_f[c].wait()

    return pl.pallas_call(
        body,
        out_shape=jax.ShapeDtypeStruct((T, D), jnp.bfloat16),
        in_specs=[
            pl.BlockSpec(memory_space=pltpu.SMEM),
            pl.BlockSpec(memory_space=pltpu.VMEM),
            pl.BlockSpec(memory_space=pl.ANY),
        ],
        out_specs=pl.BlockSpec(memory_space=pltpu.VMEM),
        scratch_shapes=[
            pltpu.VMEM((TQ, D), jnp.float32),
            pltpu.VMEM((TQ, D), jnp.bfloat16),
            pltpu.VMEM((TQ, D), jnp.bfloat16),
            pltpu.SemaphoreType.DMA((NC,)),
            pltpu.SemaphoreType.DMA((4, NC)),
            pltpu.SemaphoreType.DMA((4, NC)),
        ],
        compiler_params=pltpu.CompilerParams(collective_id=0),
    )(idx, maskf, E)


def kernel(ids, E):
    xi = lax.axis_index("x")
    yi = lax.axis_index("y")
    zi = lax.axis_index("z")
    q = 2 * xi + zi

    my_ids = lax.dynamic_slice(ids, (q * TQ,), (TQ,))
    local = my_ids - yi * V_SHARD
    in_range = (local >= 0) & (local < V_SHARD)
    idx = jnp.clip(local, 0, V_SHARD - 1)
    maskf = in_range.astype(jnp.float32)[:, None]

    return _fused(idx, maskf, E)
